# baseline (speedup 1.0000x reference)
import functools

import jax
import jax.numpy as jnp
from jax import lax
from jax.experimental import pallas as pl
from jax.experimental.pallas import tpu as pltpu
from jax.experimental.pallas import tpu_sc as plsc

N_VOCAB = 100000
HID = 128
EMB = 64
B = 1024
L = 50

_NC = 2
_NS = 16
_NW = _NC * _NS

_BN = 2048
_NBF = N_VOCAB // _BN
_TAIL = N_VOCAB - _NBF * _BN
_NO = 2
_NWS = 30
_TSLOT = _NBF - _NWS
_NX = 4


def _make_gather():
    n_idx = B * L
    per_w = n_idx // _NW
    mesh = plsc.VectorSubcoreMesh(core_axis_name="c", subcore_axis_name="s",
                                  num_cores=_NC, num_subcores=_NS)

    @functools.partial(
        pl.kernel,
        out_type=jax.ShapeDtypeStruct((n_idx, EMB), jnp.float32),
        mesh=mesh,
        scratch_types=[
            pltpu.VMEM((per_w,), jnp.int32),
            pltpu.VMEM((per_w, EMB), jnp.float32),
            pltpu.SemaphoreType.DMA,
        ],
        compiler_params=pltpu.CompilerParams(use_tc_tiling_on_sc=False),
    )
    def gather(idx_hbm, table_hbm, out_hbm, idx_v, rows_v, sem):
        wid = lax.axis_index("s") * _NC + lax.axis_index("c")
        base = wid * per_w
        pltpu.sync_copy(idx_hbm.at[pl.ds(base, per_w)], idx_v)
        pltpu.async_copy(table_hbm.at[idx_v], rows_v, sem).wait()
        pltpu.sync_copy(rows_v, out_hbm.at[pl.ds(base, per_w)])

    return gather


_gather_cache = []


def _gather(idx, table):
    if not _gather_cache:
        _gather_cache.append(_make_gather())
    return _gather_cache[0](idx, table)


def _mega_body(wih0, whh0, bi0, bh0, wih1, whh1, bi1, bh1, bfc_ref,
               emb_hbm, w_hbm, out_hbm,
               h0, c0, h1, c1, xbuf, w_buf, o_buf, o_tail,
               xsem, wsem, tsem, osem):
    def w_copy(i, slot):
        return pltpu.make_async_copy(
            w_hbm.at[pl.ds(i * _BN, _BN)], w_buf.at[slot], wsem.at[slot])

    def x_copy(t, slot):
        return pltpu.make_async_copy(
            emb_hbm.at[t], xbuf.at[slot], xsem.at[slot])

    tail_w = pltpu.make_async_copy(
        w_hbm.at[pl.ds(_NBF * _BN, _TAIL)], w_buf.at[_TSLOT, pl.ds(0, _TAIL)],
        tsem.at[0])

    for s in range(_NWS):
        w_copy(s, s).start()
    for s in range(_NX):
        x_copy(s, s).start()

    zeros = jnp.zeros((B, HID), jnp.float32)
    c0[...] = zeros
    h1[...] = zeros
    c1[...] = zeros
    b0 = bi0[...] + bh0[...]
    b1 = bi1[...] + bh1[...]
    w_ih0 = wih0[...]
    w_hh0 = whh0[...]
    w_ih1 = wih1[...]
    w_hh1 = whh1[...]

    def act(g, b):
        g = g + b
        i = jax.nn.sigmoid(g[:, 0:HID])
        f = jax.nn.sigmoid(g[:, HID:2 * HID])
        gg = jnp.tanh(g[:, 2 * HID:3 * HID])
        o = jax.nn.sigmoid(g[:, 3 * HID:4 * HID])
        return i, f, gg, o

    def dotc(a, w):
        return lax.dot_general(a, w, (((1,), (1,)), ((), ())),
                               preferred_element_type=jnp.float32)

    x_copy(0, 0).wait()
    i0, f0, g0, o0 = act(dotc(xbuf[0], w_ih0), b0)
    c_new = i0 * g0
    c0[...] = c_new
    h0[...] = o0 * jnp.tanh(c_new)

    def step(t, _):
        hp = h0[...]
        xslot = lax.rem(t, _NX)
        x_copy(t, xslot).wait()
        x = xbuf[xslot]

        @pl.when(t + _NX - 1 < L)
        def _():
            x_copy(t + _NX - 1, lax.rem(t + _NX - 1, _NX)).start()

        ga = dotc(x, w_ih0) + dotc(hp, w_hh0)
        gb = dotc(hp, w_ih1) + dotc(h1[...], w_hh1)

        ia, fa, gga, oa = act(ga, b0)
        ca = fa * c0[...] + ia * gga
        c0[...] = ca
        h0[...] = oa * jnp.tanh(ca)

        ib, fb, ggb, ob = act(gb, b1)
        cb = fb * c1[...] + ib * ggb
        c1[...] = cb
        h1[...] = ob * jnp.tanh(cb)
        return 0

    lax.fori_loop(1, L, step, 0, unroll=False)

    hp = h0[...]
    gb = dotc(hp, w_ih1) + dotc(h1[...], w_hh1)
    ib, fb, ggb, ob = act(gb, b1)
    ht = ob * jnp.tanh(fb * c1[...] + ib * ggb)

    def o_copy(i, slot):
        return pltpu.make_async_copy(
            o_buf.at[slot], out_hbm.at[:, pl.ds(i * _BN, _BN)], osem.at[slot])

    def fc_step(i, _):
        wslot = lax.rem(i, _NWS)
        oslot = lax.rem(i, _NO)
        w_copy(i, wslot).wait()

        @pl.when(i >= _NO)
        def _():
            o_copy(i - _NO, oslot).wait()

        o_buf[oslot] = (dotc(ht, w_buf[wslot])
                        + bfc_ref[:, pl.ds(i * _BN, _BN)])
        o_copy(i, oslot).start()

        @pl.when(i + _NWS < _NBF)
        def _():
            w_copy(i + _NWS, wslot).start()

        @pl.when(i == _TSLOT + 1)
        def _():
            tail_w.start()

        return 0

    lax.fori_loop(0, _NBF, fc_step, 0, unroll=False)

    pltpu.make_async_copy(
        w_hbm.at[pl.ds(_NBF * _BN, _TAIL)], w_buf.at[_TSLOT, pl.ds(0, _TAIL)],
        tsem.at[0]).wait()
    o_tail[...] = (dotc(ht, w_buf[_TSLOT, pl.ds(0, _TAIL)])
                   + bfc_ref[:, pl.ds(_NBF * _BN, _TAIL)])
    to = pltpu.make_async_copy(
        o_tail, out_hbm.at[:, pl.ds(_NBF * _BN, _TAIL)], tsem.at[0])
    to.start()
    for d in range(_NO):
        blk = _NBF - _NO + d
        pltpu.make_async_copy(
            o_buf.at[blk % _NO], out_hbm.at[:, pl.ds(blk * _BN, _BN)],
            osem.at[blk % _NO]).wait()
    to.wait()


def _mega(emb, W_ih0, W_hh0, bi0, bh0, W_ih1, W_hh1, bi1, bh1, b_fc2d, W_fc):
    return pl.pallas_call(
        _mega_body,
        in_specs=[pl.BlockSpec(memory_space=pltpu.VMEM)] * 9
        + [pl.BlockSpec(memory_space=pl.ANY),
           pl.BlockSpec(memory_space=pl.ANY)],
        out_specs=pl.BlockSpec(memory_space=pl.ANY),
        out_shape=jax.ShapeDtypeStruct((B, N_VOCAB), jnp.float32),
        scratch_shapes=[
            pltpu.VMEM((B, HID), jnp.float32),
            pltpu.VMEM((B, HID), jnp.float32),
            pltpu.VMEM((B, HID), jnp.float32),
            pltpu.VMEM((B, HID), jnp.float32),
            pltpu.VMEM((_NX, B, EMB), jnp.float32),
            pltpu.VMEM((_NWS, _BN, HID), jnp.float32),
            pltpu.VMEM((_NO, B, _BN), jnp.float32),
            pltpu.VMEM((B, _TAIL), jnp.float32),
            pltpu.SemaphoreType.DMA((_NX,)),
            pltpu.SemaphoreType.DMA((_NWS,)),
            pltpu.SemaphoreType.DMA((1,)),
            pltpu.SemaphoreType.DMA((_NO,)),
        ],
        compiler_params=pltpu.CompilerParams(
            vmem_limit_bytes=63 * 1024 * 1024),
    )(W_ih0, W_hh0, bi0, bh0, W_ih1, W_hh1, bi1, bh1, b_fc2d, emb, W_fc)


def kernel(seq_in, embeddings, W_ih0, W_hh0, b_ih0, b_hh0,
           W_ih1, W_hh1, b_ih1, b_hh1, W_fc, b_fc):
    idx = seq_in.T.reshape(-1).astype(jnp.int32)
    emb = _gather(idx, embeddings).reshape(L, B, EMB)
    return _mega(emb,
                 W_ih0, W_hh0, b_ih0.reshape(1, -1), b_hh0.reshape(1, -1),
                 W_ih1, W_hh1, b_ih1.reshape(1, -1), b_hh1.reshape(1, -1),
                 b_fc.reshape(1, -1), W_fc)


# speedup vs baseline: 1.0124x; 1.0124x over previous
import functools

import jax
import jax.numpy as jnp
from jax import lax
from jax.experimental import pallas as pl
from jax.experimental.pallas import tpu as pltpu
from jax.experimental.pallas import tpu_sc as plsc

N_VOCAB = 100000
HID = 128
EMB = 64
B = 1024
L = 50

_NC = 2
_NS = 16
_NW = _NC * _NS

_BN = 2048
_NBF = N_VOCAB // _BN
_TAIL = N_VOCAB - _NBF * _BN
_NO = 2
_NWS = 30
_TSLOT = _NBF - _NWS
_NX = 4


def _make_gather():
    n_idx = B * L
    per_w = n_idx // _NW
    mesh = plsc.VectorSubcoreMesh(core_axis_name="c", subcore_axis_name="s",
                                  num_cores=_NC, num_subcores=_NS)

    @functools.partial(
        pl.kernel,
        out_type=jax.ShapeDtypeStruct((n_idx, EMB), jnp.float32),
        mesh=mesh,
        scratch_types=[
            pltpu.VMEM((per_w,), jnp.int32),
            pltpu.VMEM((per_w, EMB), jnp.float32),
            pltpu.SemaphoreType.DMA,
        ],
        compiler_params=pltpu.CompilerParams(use_tc_tiling_on_sc=False),
    )
    def gather(idx_hbm, table_hbm, out_hbm, idx_v, rows_v, sem):
        wid = lax.axis_index("s") * _NC + lax.axis_index("c")
        base = wid * per_w
        pltpu.sync_copy(idx_hbm.at[pl.ds(base, per_w)], idx_v)
        pltpu.async_copy(table_hbm.at[idx_v], rows_v, sem).wait()
        pltpu.sync_copy(rows_v, out_hbm.at[pl.ds(base, per_w)])

    return gather


_gather_cache = []


def _gather(idx, table):
    if not _gather_cache:
        _gather_cache.append(_make_gather())
    return _gather_cache[0](idx, table)


def _mega_body(wih0, whh0, bi0, bh0, wih1, whh1, bi1, bh1, bfc_ref,
               emb_hbm, w_hbm, out_hbm,
               h0, c0, h1, c1, xbuf, w_buf, o_buf, o_tail,
               xsem, wsem, tsem, osem):
    def w_copy(i, slot):
        return pltpu.make_async_copy(
            w_hbm.at[pl.ds(i * _BN, _BN)], w_buf.at[slot], wsem.at[slot])

    def x_copy(t, slot):
        return pltpu.make_async_copy(
            emb_hbm.at[t], xbuf.at[slot], xsem.at[slot])

    tail_w = pltpu.make_async_copy(
        w_hbm.at[pl.ds(_NBF * _BN, _TAIL)], w_buf.at[_TSLOT, pl.ds(0, _TAIL)],
        tsem.at[0])

    for s in range(_NX):
        x_copy(s, s).start()
    w_copy(0, 0).start()
    w_copy(1, 1).start()

    zeros = jnp.zeros((B, HID), jnp.float32)
    c0[...] = zeros
    h1[...] = zeros
    c1[...] = zeros
    b0 = bi0[...] + bh0[...]
    b1 = bi1[...] + bh1[...]
    w_ih0 = wih0[...]
    w_hh0 = whh0[...]
    w_ih1 = wih1[...]
    w_hh1 = whh1[...]

    def act(g, b):
        g = g + b
        i = jax.nn.sigmoid(g[:, 0:HID])
        f = jax.nn.sigmoid(g[:, HID:2 * HID])
        gg = jnp.tanh(g[:, 2 * HID:3 * HID])
        o = jax.nn.sigmoid(g[:, 3 * HID:4 * HID])
        return i, f, gg, o

    def dotc(a, w):
        return lax.dot_general(a, w, (((1,), (1,)), ((), ())),
                               preferred_element_type=jnp.float32)

    x_copy(0, 0).wait()
    i0, f0, g0, o0 = act(dotc(xbuf[0], w_ih0), b0)
    c_new = i0 * g0
    c0[...] = c_new
    h0[...] = o0 * jnp.tanh(c_new)

    def step(t, _):
        hp = h0[...]
        xslot = lax.rem(t, _NX)
        x_copy(t, xslot).wait()
        x = xbuf[xslot]

        @pl.when(t + _NX - 1 < L)
        def _():
            x_copy(t + _NX - 1, lax.rem(t + _NX - 1, _NX)).start()

        @pl.when(t <= (_NWS - 2) // 2)
        def _():
            w_copy(2 * t, 2 * t).start()
            w_copy(2 * t + 1, 2 * t + 1).start()

        ga = dotc(x, w_ih0) + dotc(hp, w_hh0)
        gb = dotc(hp, w_ih1) + dotc(h1[...], w_hh1)

        ia, fa, gga, oa = act(ga, b0)
        ca = fa * c0[...] + ia * gga
        c0[...] = ca
        h0[...] = oa * jnp.tanh(ca)

        ib, fb, ggb, ob = act(gb, b1)
        cb = fb * c1[...] + ib * ggb
        c1[...] = cb
        h1[...] = ob * jnp.tanh(cb)
        return 0

    lax.fori_loop(1, L, step, 0, unroll=False)

    hp = h0[...]
    gb = dotc(hp, w_ih1) + dotc(h1[...], w_hh1)
    ib, fb, ggb, ob = act(gb, b1)
    ht = ob * jnp.tanh(fb * c1[...] + ib * ggb)

    def o_copy(i, slot):
        return pltpu.make_async_copy(
            o_buf.at[slot], out_hbm.at[:, pl.ds(i * _BN, _BN)], osem.at[slot])

    def fc_step(i, _):
        wslot = lax.rem(i, _NWS)
        oslot = lax.rem(i, _NO)
        w_copy(i, wslot).wait()

        @pl.when(i >= _NO)
        def _():
            o_copy(i - _NO, oslot).wait()

        o_buf[oslot] = (dotc(ht, w_buf[wslot])
                        + bfc_ref[:, pl.ds(i * _BN, _BN)])
        o_copy(i, oslot).start()

        @pl.when(i + _NWS < _NBF)
        def _():
            w_copy(i + _NWS, wslot).start()

        @pl.when(i == _TSLOT + 1)
        def _():
            tail_w.start()

        return 0

    lax.fori_loop(0, _NBF, fc_step, 0, unroll=False)

    pltpu.make_async_copy(
        w_hbm.at[pl.ds(_NBF * _BN, _TAIL)], w_buf.at[_TSLOT, pl.ds(0, _TAIL)],
        tsem.at[0]).wait()
    o_tail[...] = (dotc(ht, w_buf[_TSLOT, pl.ds(0, _TAIL)])
                   + bfc_ref[:, pl.ds(_NBF * _BN, _TAIL)])
    to = pltpu.make_async_copy(
        o_tail, out_hbm.at[:, pl.ds(_NBF * _BN, _TAIL)], tsem.at[0])
    to.start()
    for d in range(_NO):
        blk = _NBF - _NO + d
        pltpu.make_async_copy(
            o_buf.at[blk % _NO], out_hbm.at[:, pl.ds(blk * _BN, _BN)],
            osem.at[blk % _NO]).wait()
    to.wait()


def _mega(emb, W_ih0, W_hh0, bi0, bh0, W_ih1, W_hh1, bi1, bh1, b_fc2d, W_fc):
    return pl.pallas_call(
        _mega_body,
        in_specs=[pl.BlockSpec(memory_space=pltpu.VMEM)] * 9
        + [pl.BlockSpec(memory_space=pl.ANY),
           pl.BlockSpec(memory_space=pl.ANY)],
        out_specs=pl.BlockSpec(memory_space=pl.ANY),
        out_shape=jax.ShapeDtypeStruct((B, N_VOCAB), jnp.float32),
        scratch_shapes=[
            pltpu.VMEM((B, HID), jnp.float32),
            pltpu.VMEM((B, HID), jnp.float32),
            pltpu.VMEM((B, HID), jnp.float32),
            pltpu.VMEM((B, HID), jnp.float32),
            pltpu.VMEM((_NX, B, EMB), jnp.float32),
            pltpu.VMEM((_NWS, _BN, HID), jnp.float32),
            pltpu.VMEM((_NO, B, _BN), jnp.float32),
            pltpu.VMEM((B, _TAIL), jnp.float32),
            pltpu.SemaphoreType.DMA((_NX,)),
            pltpu.SemaphoreType.DMA((_NWS,)),
            pltpu.SemaphoreType.DMA((1,)),
            pltpu.SemaphoreType.DMA((_NO,)),
        ],
        compiler_params=pltpu.CompilerParams(
            vmem_limit_bytes=63 * 1024 * 1024),
    )(W_ih0, W_hh0, bi0, bh0, W_ih1, W_hh1, bi1, bh1, b_fc2d, emb, W_fc)


def kernel(seq_in, embeddings, W_ih0, W_hh0, b_ih0, b_hh0,
           W_ih1, W_hh1, b_ih1, b_hh1, W_fc, b_fc):
    idx = seq_in.T.reshape(-1).astype(jnp.int32)
    emb = _gather(idx, embeddings).reshape(L, B, EMB)
    return _mega(emb,
                 W_ih0, W_hh0, b_ih0.reshape(1, -1), b_hh0.reshape(1, -1),
                 W_ih1, W_hh1, b_ih1.reshape(1, -1), b_hh1.reshape(1, -1),
                 b_fc.reshape(1, -1), W_fc)
